# baseline (speedup 1.0000x reference)
import functools

import numpy as np
import jax
import jax.numpy as jnp
from jax import lax
from jax.experimental import pallas as pl
from jax.experimental.pallas import tpu as pltpu
from jax.experimental.pallas import tpu_sc as plsc

V = 10000
E = 160000
D = 32
NFREQ = 9

NC = 2
NS = 16
NW = NC * NS
CH = 128
EP = NW * 40 * CH
EPW = EP // NW
NCH = EPW // CH
ROWS_I = EP // CH
ACC_ROWS = 10240
ZPS = ACC_ROWS // NS
WPS = 624

BE = 4096
BQ = BE // 4
BV = 2000

_FT = [(0, 8), (1, 8), (2, 4)]


_AJ = (2.0 * np.pi / float(NFREQ)) * np.arange(NFREQ)


def _feat_rows(a):
    cols = [np.ones_like(a)]
    for m in (1, 2, 3, 4):
        cols += [np.cos(m * a), np.sin(m * a)]
    return np.stack(cols, axis=-1)


_FINV = np.linalg.inv(_feat_rows(_AJ)).astype(np.float32)

_JBT = np.zeros((D, D), np.float32)
for _p in list(range(8, 24, 2)) + list(range(24, 32, 2)):
    _JBT[_p + 1, _p] = -1.0
    _JBT[_p, _p + 1] = 1.0

_I2 = np.eye(2, dtype=np.float32)
_J2 = np.array([[0.0, -1.0], [1.0, 0.0]], np.float32)


def _rot_j(phi):
    c, s = jnp.cos(phi), jnp.sin(phi)
    return jnp.stack([jnp.stack([c, -s], -1), jnp.stack([s, c], -1)], -2)


def _refl_j(phi):
    c, s = jnp.cos(phi), jnp.sin(phi)
    return jnp.stack([jnp.stack([c, s], -1), jnp.stack([s, -c], -1)], -2)


def _kneigh_samples(p):
    a = jnp.asarray(_AJ, jnp.float32)
    n = a.shape[0]
    rows = []
    for n_out, m_out in _FT:
        cols = []
        for n_in, m_in in _FT:
            c = p['Kn_%d%d' % (n_out, n_in)]
            if n_out == 0 and n_in == 0:
                blk = jnp.broadcast_to(c[..., 0][None], (n, m_out, m_in))
            elif n_in == 0:
                m = n_out
                b1 = jnp.stack([jnp.cos(m * a), jnp.sin(m * a)], -1)
                b2 = jnp.stack([-jnp.sin(m * a), jnp.cos(m * a)], -1)
                basis = jnp.stack([b1, b2], -1)
                blk = jnp.einsum('ijk,edk->eidj', c, basis, precision=lax.Precision.HIGHEST).reshape(n, m_out * 2, m_in)
            elif n_out == 0:
                q = n_in
                r1 = jnp.stack([jnp.cos(q * a), jnp.sin(q * a)], -1)
                r2 = jnp.stack([jnp.sin(q * a), -jnp.cos(q * a)], -1)
                basis = jnp.stack([r1, r2], -1)
                blk = jnp.einsum('ijk,edk->eijd', c, basis, precision=lax.Precision.HIGHEST).reshape(n, m_out, m_in * 2)
            else:
                B1 = _rot_j((n_out - n_in) * a)
                B2 = jnp.einsum('ab,ebc->eac', jnp.asarray(_J2), B1, precision=lax.Precision.HIGHEST)
                B3 = _refl_j((n_out + n_in) * a)
                B4 = jnp.einsum('ab,ebc->eac', jnp.asarray(_J2), B3, precision=lax.Precision.HIGHEST)
                basis = jnp.stack([B1, B2, B3, B4], -1)
                blk = jnp.einsum('ijk,eabk->eiajb', c, basis, precision=lax.Precision.HIGHEST).reshape(n, m_out * 2, m_in * 2)
            cols.append(blk)
        rows.append(jnp.concatenate(cols, axis=2))
    return jnp.concatenate(rows, axis=1)


def _kself_mat(p):
    rows = []
    for n_out, m_out in _FT:
        d_out = 1 if n_out == 0 else 2
        cols = []
        for n_in, m_in in _FT:
            d_in = 1 if n_in == 0 else 2
            if n_in != n_out:
                cols.append(jnp.zeros((m_out * d_out, m_in * d_in), jnp.float32))
            elif n_out == 0:
                cols.append(p['Kself_0'])
            else:
                c = p['Kself_%d' % n_out]
                mats = (c[..., 0, None, None] * jnp.asarray(_I2)
                        + c[..., 1, None, None] * jnp.asarray(_J2))
                cols.append(jnp.transpose(mats, (0, 2, 1, 3)).reshape(m_out * 2, m_in * 2))
        rows.append(jnp.concatenate(cols, axis=1))
    return jnp.concatenate(rows, axis=0)


def _build_ct(p):
    k9 = _kneigh_samples(p)
    c = jnp.einsum('kj,joi->koi', jnp.asarray(_FINV), k9,
                   precision=lax.Precision.HIGHEST)
    return jnp.transpose(c, (0, 2, 1)).reshape(NFREQ * D, D)


@functools.cache
def _get_sc_gather(nch):
    epw = nch * CH
    mesh = plsc.VectorSubcoreMesh(core_axis_name="c", subcore_axis_name="s")

    @functools.partial(
        pl.kernel, mesh=mesh,
        compiler_params=pltpu.CompilerParams(use_tc_tiling_on_sc=False),
        out_type=jax.ShapeDtypeStruct((epw * NW, D), jnp.float32),
        scratch_types=[
            pltpu.VMEM((nch, CH), jnp.int32),
            pltpu.VMEM((2, CH, D), jnp.float32),
            pltpu.SemaphoreType.DMA,
            pltpu.SemaphoreType.DMA,
            pltpu.SemaphoreType.DMA,
            pltpu.SemaphoreType.DMA,
        ],
    )
    def _sc_gather(x_hbm, src_hbm, out_hbm, idx_v, rows_v, gs0, gs1, ss0, ss1):
        wid = lax.axis_index("s") * NC + lax.axis_index("c")
        pltpu.sync_copy(src_hbm.at[pl.ds(wid * nch, nch)], idx_v)
        gsems = [gs0, gs1]
        ssems = [ss0, ss1]
        gops = [None, None]
        sops = [None, None]
        gops[0] = pltpu.async_copy(x_hbm.at[idx_v.at[0]], rows_v.at[0], gsems[0])
        for j in range(nch):
            b = j & 1
            nb = 1 - b
            gops[b].wait()
            if j + 1 < nch:
                if sops[nb] is not None:
                    sops[nb].wait()
                gops[nb] = pltpu.async_copy(
                    x_hbm.at[idx_v.at[j + 1]], rows_v.at[nb], gsems[nb])
            sops[b] = pltpu.async_copy(
                rows_v.at[b], out_hbm.at[pl.ds(wid * epw + j * CH, CH)], ssems[b])
        sops[0].wait()
        sops[1].wait()

    return _sc_gather


_PHIGH = lax.Precision.HIGHEST


def _msg_body(fq_ref, an_ref, tr_ref, jbtt_ref, ct_ref, out_ref):
    xb = fq_ref[...]
    xt4 = jnp.transpose(xb)
    tq = tr_ref[0]
    aq = an_ref[0]
    ct1 = jnp.cos(tq)
    st1 = jnp.sin(tq)
    ct2 = 2.0 * ct1 * ct1 - 1.0
    st2 = 2.0 * st1 * ct1
    c1 = jnp.cos(aq)
    s1 = jnp.sin(aq)
    c2 = 2.0 * c1 * c1 - 1.0
    s2 = 2.0 * s1 * c1
    c3 = c2 * c1 - s2 * s1
    s3 = s2 * c1 + c2 * s1
    c4 = c3 * c1 - s3 * s1
    s4 = s3 * c1 + c3 * s1
    jbtt = jbtt_ref[...]
    ctm = ct_ref[...]
    one8 = jnp.ones((8, BQ), jnp.float32)
    zero8 = jnp.zeros((8, BQ), jnp.float32)
    outs = []
    for q in range(4):
        xt = xt4[32 * q:32 * (q + 1), :]
        xjt = jnp.dot(jbtt, xt,
                      preferred_element_type=jnp.float32, precision=_PHIGH)
        w0 = jnp.concatenate([one8,
                              jnp.broadcast_to(ct1[q:q + 1], (16, BQ)),
                              jnp.broadcast_to(ct2[q:q + 1], (8, BQ))], axis=0)
        w1 = jnp.concatenate([zero8,
                              jnp.broadcast_to(st1[q:q + 1], (16, BQ)),
                              jnp.broadcast_to(st2[q:q + 1], (8, BQ))], axis=0)
        y = w0 * xt + w1 * xjt
        yhat = jnp.concatenate(
            [y, c1[q:q + 1] * y, s1[q:q + 1] * y, c2[q:q + 1] * y,
             s2[q:q + 1] * y, c3[q:q + 1] * y, s3[q:q + 1] * y,
             c4[q:q + 1] * y, s4[q:q + 1] * y], axis=0)
        outs.append(jnp.dot(ctm, yhat,
                            preferred_element_type=jnp.float32,
                            precision=_PHIGH))
    out_ref[...] = jnp.transpose(jnp.concatenate(outs, axis=0))


def _compute_msg(fq, an4, tr4, jbtt, ct):
    fq4 = fq.reshape(fq.shape[0] // 4, 128)
    ne4 = fq4.shape[0]
    msg4 = pl.pallas_call(
        _msg_body,
        grid=(ne4 // BQ,),
        in_specs=[
            pl.BlockSpec((BQ, 128), lambda i: (i, 0)),
            pl.BlockSpec((1, 4, BQ), lambda i: (i, 0, 0)),
            pl.BlockSpec((1, 4, BQ), lambda i: (i, 0, 0)),
            pl.BlockSpec((D, D), lambda i: (0, 0)),
            pl.BlockSpec((D, NFREQ * D), lambda i: (0, 0)),
        ],
        out_specs=pl.BlockSpec((BQ, 128), lambda i: (i, 0)),
        out_shape=jax.ShapeDtypeStruct((ne4, 128), jnp.float32),
    )(fq4, an4, tr4, jbtt, ct)
    return msg4.reshape(ne4 * 4, D)


@functools.cache
def _get_sc_scatter(nch):
    epw = nch * CH
    mesh = plsc.VectorSubcoreMesh(core_axis_name="c", subcore_axis_name="s")

    @functools.partial(
        pl.kernel, mesh=mesh,
        compiler_params=pltpu.CompilerParams(use_tc_tiling_on_sc=False),
        out_type=jax.ShapeDtypeStruct((NC, V, D), jnp.float32),
        scratch_types=[
            pltpu.VMEM((nch, CH), jnp.int32),
            pltpu.VMEM((2, CH, D), jnp.float32),
            pltpu.VMEM((CH, D), jnp.float32),
            pltpu.VMEM_SHARED((ACC_ROWS, D), jnp.float32),
            pltpu.SemaphoreType.DMA,
            pltpu.SemaphoreType.DMA,
        ],
    )
    def _sc_scatter(msg_hbm, tgt_hbm, out_hbm, idx_v, rows_v, zero_v, acc_sh,
                    ls0, ls1):
        cid = lax.axis_index("c")
        sid = lax.axis_index("s")
        wid = sid * NC + cid

        def zrow(i, carry):
            zero_v[i, pl.ds(0, 16)] = jnp.zeros((16,), jnp.float32)
            zero_v[i, pl.ds(16, 16)] = jnp.zeros((16,), jnp.float32)
            return carry

        lax.fori_loop(0, CH, zrow, 0)

        def zcp(q, carry):
            pltpu.sync_copy(zero_v, acc_sh.at[pl.ds(sid * ZPS + q * CH, CH)])
            return carry

        lax.fori_loop(0, ZPS // CH, zcp, 0)
        plsc.subcore_barrier()

        pltpu.sync_copy(tgt_hbm.at[pl.ds(wid * nch, nch)], idx_v)
        lsems = [ls0, ls1]
        lops = [None, None]
        lops[0] = pltpu.async_copy(
            msg_hbm.at[pl.ds(wid * epw, CH)], rows_v.at[0], lsems[0])
        for j in range(nch):
            b = j & 1
            nb = 1 - b
            lops[b].wait()
            if j + 1 < nch:
                lops[nb] = pltpu.async_copy(
                    msg_hbm.at[pl.ds(wid * epw + (j + 1) * CH, CH)],
                    rows_v.at[nb], lsems[nb])
            pltpu.sync_copy(rows_v.at[b], acc_sh.at[idx_v.at[j]], add=True)
        plsc.subcore_barrier()

        @pl.when(sid < NS - 1)
        def _():
            pltpu.sync_copy(acc_sh.at[pl.ds(sid * WPS, WPS)],
                            out_hbm.at[cid, pl.ds(sid * WPS, WPS)])

        @pl.when(sid == NS - 1)
        def _():
            pltpu.sync_copy(acc_sh.at[pl.ds((NS - 1) * WPS, V - (NS - 1) * WPS)],
                            out_hbm.at[cid, pl.ds((NS - 1) * WPS, V - (NS - 1) * WPS)])

    return _sc_scatter


VQ = V * D // 128


def _combine_body(x_ref, ks_ref, acca_ref, accb_ref, out_ref):
    xt4 = jnp.transpose(x_ref[...])
    ks = ks_ref[...]
    selfs = []
    for q in range(4):
        selfs.append(jnp.dot(ks, xt4[32 * q:32 * (q + 1), :],
                             preferred_element_type=jnp.float32,
                             precision=_PHIGH))
    asum = (acca_ref[0] + acca_ref[1] + accb_ref[0] + accb_ref[1])
    out_ref[...] = asum + jnp.transpose(jnp.concatenate(selfs, axis=0))


def _combine(x, ks, accs):
    x4 = x.reshape(VQ, 128)
    accs4 = [a.reshape(NC, VQ, 128) for a in accs]
    out4 = pl.pallas_call(
        _combine_body,
        grid=(1,),
        in_specs=[
            pl.BlockSpec((VQ, 128), lambda i: (0, 0)),
            pl.BlockSpec((D, D), lambda i: (0, 0)),
            pl.BlockSpec((NC, VQ, 128), lambda i: (0, 0, 0)),
            pl.BlockSpec((NC, VQ, 128), lambda i: (0, 0, 0)),
        ],
        out_specs=pl.BlockSpec((VQ, 128), lambda i: (0, 0)),
        out_shape=jax.ShapeDtypeStruct((VQ, 128), jnp.float32),
    )(x4, ks, *accs4)
    return out4.reshape(V, D)


def kernel(x, edge_index, angles, transporters, Kself_0, Kself_1, Kself_2,
           Kn_00, Kn_01, Kn_02, Kn_10, Kn_11, Kn_12, Kn_20, Kn_21, Kn_22):
    p = {'Kself_0': Kself_0, 'Kself_1': Kself_1, 'Kself_2': Kself_2,
         'Kn_00': Kn_00, 'Kn_01': Kn_01, 'Kn_02': Kn_02,
         'Kn_10': Kn_10, 'Kn_11': Kn_11, 'Kn_12': Kn_12,
         'Kn_20': Kn_20, 'Kn_21': Kn_21, 'Kn_22': Kn_22}
    ct = jnp.transpose(_build_ct(p))
    ks = _kself_mat(p)
    jbtt = jnp.asarray(_JBT.T)
    pad = EP - E
    def _perm(a):
        return jnp.transpose(a.reshape(EP // BE, 4, BQ), (0, 2, 1)).reshape(-1)

    src2 = _perm(jnp.concatenate(
        [edge_index[0], jnp.zeros((pad,), jnp.int32)])).reshape(ROWS_I, CH)
    tgt2 = _perm(jnp.concatenate(
        [edge_index[1], jnp.full((pad,), V, jnp.int32)])).reshape(ROWS_I, CH)
    zf = jnp.zeros((pad,), jnp.float32)
    an4 = jnp.concatenate([angles, zf]).reshape(EP // BE, 4, BQ)
    tr4 = jnp.concatenate([transporters, zf]).reshape(EP // BE, 4, BQ)

    nseg = 2
    rows_s = ROWS_I // nseg
    nchs = NCH // nseg
    gseg = (EP // nseg) // BE
    accs = []
    for g in range(nseg):
        fq_g = _get_sc_gather(nchs)(x, src2[g * rows_s:(g + 1) * rows_s])
        msg_g = _compute_msg(fq_g, an4[g * gseg:(g + 1) * gseg],
                             tr4[g * gseg:(g + 1) * gseg], jbtt, ct)
        accs.append(_get_sc_scatter(nchs)(msg_g, tgt2[g * rows_s:(g + 1) * rows_s]))
    return _combine(x, ks, accs)


# speedup vs baseline: 1.1224x; 1.1224x over previous
import functools

import numpy as np
import jax
import jax.numpy as jnp
from jax import lax
from jax.experimental import pallas as pl
from jax.experimental.pallas import tpu as pltpu
from jax.experimental.pallas import tpu_sc as plsc

V = 10000
E = 160000
D = 32
NFREQ = 9

NC = 2
NS = 16
NW = NC * NS
CH = 128
EP = NW * 40 * CH
EPW = EP // NW
NCH = EPW // CH
ROWS_I = EP // CH
ACC_ROWS = 10240
ZPS = ACC_ROWS // NS
WPS = 624

BE = 4096
BQ = BE // 4
BV = 2000

_FT = [(0, 8), (1, 8), (2, 4)]


_AJ = (2.0 * np.pi / float(NFREQ)) * np.arange(NFREQ)


def _feat_rows(a):
    cols = [np.ones_like(a)]
    for m in (1, 2, 3, 4):
        cols += [np.cos(m * a), np.sin(m * a)]
    return np.stack(cols, axis=-1)


_FINV = np.linalg.inv(_feat_rows(_AJ)).astype(np.float32)

_JBT = np.zeros((D, D), np.float32)
for _p in list(range(8, 24, 2)) + list(range(24, 32, 2)):
    _JBT[_p + 1, _p] = -1.0
    _JBT[_p, _p + 1] = 1.0

_I2 = np.eye(2, dtype=np.float32)
_J2 = np.array([[0.0, -1.0], [1.0, 0.0]], np.float32)


def _rot_j(phi):
    c, s = jnp.cos(phi), jnp.sin(phi)
    return jnp.stack([jnp.stack([c, -s], -1), jnp.stack([s, c], -1)], -2)


def _refl_j(phi):
    c, s = jnp.cos(phi), jnp.sin(phi)
    return jnp.stack([jnp.stack([c, s], -1), jnp.stack([s, -c], -1)], -2)


def _kneigh_samples(p):
    a = jnp.asarray(_AJ, jnp.float32)
    n = a.shape[0]
    rows = []
    for n_out, m_out in _FT:
        cols = []
        for n_in, m_in in _FT:
            c = p['Kn_%d%d' % (n_out, n_in)]
            if n_out == 0 and n_in == 0:
                blk = jnp.broadcast_to(c[..., 0][None], (n, m_out, m_in))
            elif n_in == 0:
                m = n_out
                b1 = jnp.stack([jnp.cos(m * a), jnp.sin(m * a)], -1)
                b2 = jnp.stack([-jnp.sin(m * a), jnp.cos(m * a)], -1)
                basis = jnp.stack([b1, b2], -1)
                blk = jnp.einsum('ijk,edk->eidj', c, basis, precision=lax.Precision.HIGHEST).reshape(n, m_out * 2, m_in)
            elif n_out == 0:
                q = n_in
                r1 = jnp.stack([jnp.cos(q * a), jnp.sin(q * a)], -1)
                r2 = jnp.stack([jnp.sin(q * a), -jnp.cos(q * a)], -1)
                basis = jnp.stack([r1, r2], -1)
                blk = jnp.einsum('ijk,edk->eijd', c, basis, precision=lax.Precision.HIGHEST).reshape(n, m_out, m_in * 2)
            else:
                B1 = _rot_j((n_out - n_in) * a)
                B2 = jnp.einsum('ab,ebc->eac', jnp.asarray(_J2), B1, precision=lax.Precision.HIGHEST)
                B3 = _refl_j((n_out + n_in) * a)
                B4 = jnp.einsum('ab,ebc->eac', jnp.asarray(_J2), B3, precision=lax.Precision.HIGHEST)
                basis = jnp.stack([B1, B2, B3, B4], -1)
                blk = jnp.einsum('ijk,eabk->eiajb', c, basis, precision=lax.Precision.HIGHEST).reshape(n, m_out * 2, m_in * 2)
            cols.append(blk)
        rows.append(jnp.concatenate(cols, axis=2))
    return jnp.concatenate(rows, axis=1)


def _kself_mat(p):
    rows = []
    for n_out, m_out in _FT:
        d_out = 1 if n_out == 0 else 2
        cols = []
        for n_in, m_in in _FT:
            d_in = 1 if n_in == 0 else 2
            if n_in != n_out:
                cols.append(jnp.zeros((m_out * d_out, m_in * d_in), jnp.float32))
            elif n_out == 0:
                cols.append(p['Kself_0'])
            else:
                c = p['Kself_%d' % n_out]
                mats = (c[..., 0, None, None] * jnp.asarray(_I2)
                        + c[..., 1, None, None] * jnp.asarray(_J2))
                cols.append(jnp.transpose(mats, (0, 2, 1, 3)).reshape(m_out * 2, m_in * 2))
        rows.append(jnp.concatenate(cols, axis=1))
    return jnp.concatenate(rows, axis=0)


def _build_ct(p):
    k9 = _kneigh_samples(p)
    c = jnp.einsum('kj,joi->koi', jnp.asarray(_FINV), k9,
                   precision=lax.Precision.HIGHEST)
    return jnp.transpose(c, (0, 2, 1)).reshape(NFREQ * D, D)


@functools.cache
def _get_sc_gather(nch):
    epw = nch * CH
    mesh = plsc.VectorSubcoreMesh(core_axis_name="c", subcore_axis_name="s")

    @functools.partial(
        pl.kernel, mesh=mesh,
        compiler_params=pltpu.CompilerParams(use_tc_tiling_on_sc=False),
        out_type=jax.ShapeDtypeStruct((epw * NW, D), jnp.float32),
        scratch_types=[
            pltpu.VMEM((nch, CH), jnp.int32),
            pltpu.VMEM((2, CH, D), jnp.float32),
            pltpu.SemaphoreType.DMA,
            pltpu.SemaphoreType.DMA,
            pltpu.SemaphoreType.DMA,
            pltpu.SemaphoreType.DMA,
        ],
    )
    def _sc_gather(x_hbm, src_hbm, out_hbm, idx_v, rows_v, gs0, gs1, ss0, ss1):
        wid = lax.axis_index("s") * NC + lax.axis_index("c")
        pltpu.sync_copy(src_hbm.at[pl.ds(wid * nch, nch)], idx_v)
        gsems = [gs0, gs1]
        ssems = [ss0, ss1]
        gops = [None, None]
        sops = [None, None]
        gops[0] = pltpu.async_copy(x_hbm.at[idx_v.at[0]], rows_v.at[0], gsems[0])
        for j in range(nch):
            b = j & 1
            nb = 1 - b
            gops[b].wait()
            if j + 1 < nch:
                if sops[nb] is not None:
                    sops[nb].wait()
                gops[nb] = pltpu.async_copy(
                    x_hbm.at[idx_v.at[j + 1]], rows_v.at[nb], gsems[nb])
            sops[b] = pltpu.async_copy(
                rows_v.at[b], out_hbm.at[pl.ds(wid * epw + j * CH, CH)], ssems[b])
        sops[0].wait()
        sops[1].wait()

    return _sc_gather


_PHIGH = lax.Precision.HIGHEST


def _msg_body(fq_ref, an_ref, tr_ref, jbtt_ref, ct_ref, out_ref):
    xb = fq_ref[...]
    xt4 = jnp.transpose(xb)
    tq = tr_ref[0]
    aq = an_ref[0]
    ct1 = jnp.cos(tq)
    st1 = jnp.sin(tq)
    ct2 = 2.0 * ct1 * ct1 - 1.0
    st2 = 2.0 * st1 * ct1
    c1 = jnp.cos(aq)
    s1 = jnp.sin(aq)
    c2 = 2.0 * c1 * c1 - 1.0
    s2 = 2.0 * s1 * c1
    c3 = c2 * c1 - s2 * s1
    s3 = s2 * c1 + c2 * s1
    c4 = c3 * c1 - s3 * s1
    s4 = s3 * c1 + c3 * s1
    jbtt = jbtt_ref[...]
    ctm = ct_ref[...]
    one8 = jnp.ones((8, BQ), jnp.float32)
    zero8 = jnp.zeros((8, BQ), jnp.float32)
    outs = []
    for q in range(4):
        xt = xt4[32 * q:32 * (q + 1), :]
        xjt = jnp.dot(jbtt, xt,
                      preferred_element_type=jnp.float32, precision=_PHIGH)
        w0 = jnp.concatenate([one8,
                              jnp.broadcast_to(ct1[q:q + 1], (16, BQ)),
                              jnp.broadcast_to(ct2[q:q + 1], (8, BQ))], axis=0)
        w1 = jnp.concatenate([zero8,
                              jnp.broadcast_to(st1[q:q + 1], (16, BQ)),
                              jnp.broadcast_to(st2[q:q + 1], (8, BQ))], axis=0)
        y = w0 * xt + w1 * xjt
        yhat = jnp.concatenate(
            [y, c1[q:q + 1] * y, s1[q:q + 1] * y, c2[q:q + 1] * y,
             s2[q:q + 1] * y, c3[q:q + 1] * y, s3[q:q + 1] * y,
             c4[q:q + 1] * y, s4[q:q + 1] * y], axis=0)
        outs.append(jnp.dot(ctm, yhat,
                            preferred_element_type=jnp.float32,
                            precision=_PHIGH))
    out_ref[...] = jnp.transpose(jnp.concatenate(outs, axis=0))


def _compute_msg(fq, an4, tr4, jbtt, ct):
    fq4 = fq.reshape(fq.shape[0] // 4, 128)
    ne4 = fq4.shape[0]
    msg4 = pl.pallas_call(
        _msg_body,
        grid=(ne4 // BQ,),
        in_specs=[
            pl.BlockSpec((BQ, 128), lambda i: (i, 0)),
            pl.BlockSpec((1, 4, BQ), lambda i: (i, 0, 0)),
            pl.BlockSpec((1, 4, BQ), lambda i: (i, 0, 0)),
            pl.BlockSpec((D, D), lambda i: (0, 0)),
            pl.BlockSpec((D, NFREQ * D), lambda i: (0, 0)),
        ],
        out_specs=pl.BlockSpec((BQ, 128), lambda i: (i, 0)),
        out_shape=jax.ShapeDtypeStruct((ne4, 128), jnp.float32),
    )(fq4, an4, tr4, jbtt, ct)
    return msg4.reshape(ne4 * 4, D)


@functools.cache
def _get_sc_scatter(nch):
    epw = nch * CH
    mesh = plsc.VectorSubcoreMesh(core_axis_name="c", subcore_axis_name="s")

    @functools.partial(
        pl.kernel, mesh=mesh,
        compiler_params=pltpu.CompilerParams(use_tc_tiling_on_sc=False),
        out_type=jax.ShapeDtypeStruct((NC, V, D), jnp.float32),
        scratch_types=[
            pltpu.VMEM((nch, CH), jnp.int32),
            pltpu.VMEM((2, CH, D), jnp.float32),
            pltpu.VMEM((CH, D), jnp.float32),
            pltpu.VMEM_SHARED((ACC_ROWS, D), jnp.float32),
            pltpu.SemaphoreType.DMA,
            pltpu.SemaphoreType.DMA,
        ],
    )
    def _sc_scatter(msg_hbm, tgt_hbm, out_hbm, idx_v, rows_v, zero_v, acc_sh,
                    ls0, ls1):
        cid = lax.axis_index("c")
        sid = lax.axis_index("s")
        wid = sid * NC + cid

        def zrow(i, carry):
            zero_v[i, pl.ds(0, 16)] = jnp.zeros((16,), jnp.float32)
            zero_v[i, pl.ds(16, 16)] = jnp.zeros((16,), jnp.float32)
            return carry

        lax.fori_loop(0, CH, zrow, 0)

        def zcp(q, carry):
            pltpu.sync_copy(zero_v, acc_sh.at[pl.ds(sid * ZPS + q * CH, CH)])
            return carry

        lax.fori_loop(0, ZPS // CH, zcp, 0)
        plsc.subcore_barrier()

        pltpu.sync_copy(tgt_hbm.at[pl.ds(wid * nch, nch)], idx_v)
        lsems = [ls0, ls1]
        lops = [None, None]
        lops[0] = pltpu.async_copy(
            msg_hbm.at[pl.ds(wid * epw, CH)], rows_v.at[0], lsems[0])
        for j in range(nch):
            b = j & 1
            nb = 1 - b
            lops[b].wait()
            if j + 1 < nch:
                lops[nb] = pltpu.async_copy(
                    msg_hbm.at[pl.ds(wid * epw + (j + 1) * CH, CH)],
                    rows_v.at[nb], lsems[nb])
            pltpu.sync_copy(rows_v.at[b], acc_sh.at[idx_v.at[j]], add=True)
        plsc.subcore_barrier()

        @pl.when(sid < NS - 1)
        def _():
            pltpu.sync_copy(acc_sh.at[pl.ds(sid * WPS, WPS)],
                            out_hbm.at[cid, pl.ds(sid * WPS, WPS)])

        @pl.when(sid == NS - 1)
        def _():
            pltpu.sync_copy(acc_sh.at[pl.ds((NS - 1) * WPS, V - (NS - 1) * WPS)],
                            out_hbm.at[cid, pl.ds((NS - 1) * WPS, V - (NS - 1) * WPS)])

    return _sc_scatter


VQ = V * D // 128


def _combine_body(x_ref, ks_ref, acca_ref, accb_ref, out_ref):
    xt4 = jnp.transpose(x_ref[...])
    ks = ks_ref[...]
    selfs = []
    for q in range(4):
        selfs.append(jnp.dot(ks, xt4[32 * q:32 * (q + 1), :],
                             preferred_element_type=jnp.float32,
                             precision=_PHIGH))
    asum = (acca_ref[0] + acca_ref[1] + accb_ref[0] + accb_ref[1])
    out_ref[...] = asum + jnp.transpose(jnp.concatenate(selfs, axis=0))


def _combine(x, ks, accs):
    x4 = x.reshape(VQ, 128)
    accs4 = [a.reshape(NC, VQ, 128) for a in accs]
    out4 = pl.pallas_call(
        _combine_body,
        grid=(1,),
        in_specs=[
            pl.BlockSpec((VQ, 128), lambda i: (0, 0)),
            pl.BlockSpec((D, D), lambda i: (0, 0)),
            pl.BlockSpec((NC, VQ, 128), lambda i: (0, 0, 0)),
            pl.BlockSpec((NC, VQ, 128), lambda i: (0, 0, 0)),
        ],
        out_specs=pl.BlockSpec((VQ, 128), lambda i: (0, 0)),
        out_shape=jax.ShapeDtypeStruct((VQ, 128), jnp.float32),
    )(x4, ks, *accs4)
    return out4.reshape(V, D)


def kernel(x, edge_index, angles, transporters, Kself_0, Kself_1, Kself_2,
           Kn_00, Kn_01, Kn_02, Kn_10, Kn_11, Kn_12, Kn_20, Kn_21, Kn_22):
    p = {'Kself_0': Kself_0, 'Kself_1': Kself_1, 'Kself_2': Kself_2,
         'Kn_00': Kn_00, 'Kn_01': Kn_01, 'Kn_02': Kn_02,
         'Kn_10': Kn_10, 'Kn_11': Kn_11, 'Kn_12': Kn_12,
         'Kn_20': Kn_20, 'Kn_21': Kn_21, 'Kn_22': Kn_22}
    ct = jnp.transpose(_build_ct(p))
    ks = _kself_mat(p)
    jbtt = jnp.asarray(_JBT.T)
    pad = EP - E
    src2 = jnp.concatenate(
        [edge_index[0], jnp.zeros((pad,), jnp.int32)]).reshape(ROWS_I, CH)
    tgt2 = jnp.concatenate(
        [edge_index[1], jnp.full((pad,), V, jnp.int32)]).reshape(ROWS_I, CH)
    zf = jnp.zeros((pad,), jnp.float32)
    an4 = jnp.transpose(
        jnp.concatenate([angles, zf]).reshape(EP // BE, BQ, 4), (0, 2, 1))
    tr4 = jnp.transpose(
        jnp.concatenate([transporters, zf]).reshape(EP // BE, BQ, 4), (0, 2, 1))

    nseg = 2
    rows_s = ROWS_I // nseg
    nchs = NCH // nseg
    gseg = (EP // nseg) // BE
    accs = []
    for g in range(nseg):
        fq_g = _get_sc_gather(nchs)(x, src2[g * rows_s:(g + 1) * rows_s])
        msg_g = _compute_msg(fq_g, an4[g * gseg:(g + 1) * gseg],
                             tr4[g * gseg:(g + 1) * gseg], jbtt, ct)
        accs.append(_get_sc_scatter(nchs)(msg_g, tgt2[g * rows_s:(g + 1) * rows_s]))
    return _combine(x, ks, accs)
